# baseline (speedup 1.0000x reference)
import functools

import jax
import jax.numpy as jnp
from jax import lax
from jax.experimental import pallas as pl
from jax.experimental.pallas import tpu as pltpu
from jax.experimental.pallas import tpu_sc as plsc

K = 16
TS = 256
BIG = 3.0e38


def _topk_body(xyz_ref, sxyzT_ref, sxyz_ref, idx_ref, gxn_ref):
    b = pl.program_id(0)
    x = xyz_ref[0]
    sT = sxyzT_ref[0]
    smat = sxyz_ref[0]
    n = sT.shape[1]
    s0, s1, s2 = sT[0:1, :], sT[1:2, :], sT[2:3, :]
    x0, x1, x2 = x[:, 0:1], x[:, 1:2], x[:, 2:3]
    ssq = (s0 * s0 + s1 * s1) + s2 * s2
    xsq = (x0 * x0 + x1 * x1) + x2 * x2
    mm = lax.dot_general(
        x.astype(jnp.bfloat16), sT.astype(jnp.bfloat16),
        (((1,), (0,)), ((), ())), preferred_element_type=jnp.float32)
    d = (-2.0 * mm + xsq) + ssq
    iota = lax.broadcasted_iota(jnp.int32, (TS, n), 1)
    smat_hi = smat.astype(jnp.bfloat16)
    smat_lo = (smat - smat_hi.astype(jnp.float32)).astype(jnp.bfloat16)
    iota_n = lax.broadcasted_iota(jnp.int32, (n, 1), 0)
    ext = jnp.concatenate([
        smat_hi, smat_lo,
        (iota_n // 64).astype(jnp.bfloat16),
        (iota_n % 64).astype(jnp.bfloat16),
        jnp.full((n, 1), 1.0, jnp.bfloat16),
    ], axis=1)
    d0 = d
    sels, gxs = [], []
    tiecnt = jnp.zeros((TS, 1), jnp.float32)
    for _ in range(K):
        m = jnp.min(d, axis=1, keepdims=True)
        eq = d == m
        onehot = jnp.where(eq, 1.0, 0.0).astype(jnp.bfloat16)
        ghl = lax.dot_general(onehot, ext, (((1,), (0,)), ((), ())),
                              preferred_element_type=jnp.float32)
        sel = (ghl[:, 6:7] * 64.0 + ghl[:, 7:8]).astype(jnp.int32)
        tiecnt = jnp.maximum(tiecnt, ghl[:, 8:9])
        sels.append(sel)
        gxs.append((ghl[:, 0:3] + ghl[:, 3:6]) - x)
        d = jnp.where(iota == sel, BIG, d)
    idx_fast = jnp.concatenate(sels, axis=1)
    gxn_fast = jnp.concatenate(gxs, axis=1)

    def _exact_topk():
        dd = d0
        sels_e, gxs_e = [], []
        for _ in range(K):
            me = jnp.min(dd, axis=1, keepdims=True)
            cand = jnp.where(dd == me, iota, n)
            sel_e = jnp.min(cand, axis=1, keepdims=True)
            hit = iota == sel_e
            gx_e = jnp.concatenate(
                [jnp.sum(jnp.where(hit, sc, 0.0), axis=1, keepdims=True)
                 for sc in (s0, s1, s2)], axis=1)
            sels_e.append(sel_e)
            gxs_e.append(gx_e - x)
            dd = jnp.where(hit, BIG, dd)
        return (jnp.concatenate(sels_e, axis=1),
                jnp.concatenate(gxs_e, axis=1))

    idx, gxn = lax.cond(jnp.max(tiecnt) >= 1.5, _exact_topk,
                        lambda: (idx_fast, gxn_fast))
    idx_ref[0] = idx + b * n
    gxn_ref[0] = gxn


def _tc_topk(xyz, s_xyzT, s_xyz):
    B, S, _ = xyz.shape
    N = s_xyzT.shape[2]
    return pl.pallas_call(
        _topk_body,
        grid=(B, S // TS),
        in_specs=[
            pl.BlockSpec((1, TS, 3), lambda b, t: (b, t, 0)),
            pl.BlockSpec((1, 3, N), lambda b, t: (b, 0, 0)),
            pl.BlockSpec((1, N, 3), lambda b, t: (b, 0, 0)),
        ],
        out_specs=[
            pl.BlockSpec((1, TS, K), lambda b, t: (b, t, 0)),
            pl.BlockSpec((1, TS, K * 3), lambda b, t: (b, t, 0)),
        ],
        out_shape=[
            jax.ShapeDtypeStruct((B, S, K), jnp.int32),
            jax.ShapeDtypeStruct((B, S, K * 3), jnp.float32),
        ],
    )(xyz, s_xyzT, s_xyz)


def _sc_gather(idx_flat, pts_tab):
    QK = idx_flat.shape[0]
    D = pts_tab.shape[1]
    info = plsc.get_sparse_core_info()
    NC, NS = info.num_cores, info.num_subcores
    NW = NC * NS
    per_w = QK // NW
    G = 128
    ngroups = per_w // G
    mesh = plsc.VectorSubcoreMesh(core_axis_name="c", subcore_axis_name="s")

    @functools.partial(
        pl.kernel,
        out_type=jax.ShapeDtypeStruct((QK, D), jnp.float32),
        mesh=mesh,
        scratch_types=[
            pltpu.VMEM((2, G), jnp.int32),
            pltpu.VMEM((2, G, D), jnp.float32),
            pltpu.SemaphoreType.DMA,
            pltpu.SemaphoreType.DMA,
            pltpu.SemaphoreType.DMA,
        ],
    )
    def body(idx_hbm, pts_hbm, outp_hbm, idx_v, pts_v, sem0, sem1, osem):
        wid = lax.axis_index("s") * NC + lax.axis_index("c")
        base = wid * per_w

        def start_gather(g, slot, sem):
            pltpu.sync_copy(idx_hbm.at[pl.ds(base + g * G, G)],
                            idx_v.at[slot])
            pltpu.async_copy(pts_hbm.at[idx_v.at[slot]], pts_v.at[slot], sem)

        def drain_writeback(g, slot, sem):
            pltpu.make_async_copy(pts_hbm.at[idx_v.at[slot]],
                                  pts_v.at[slot], sem).wait()
            pltpu.async_copy(pts_v.at[slot],
                             outp_hbm.at[pl.ds(base + g * G, G)], osem).wait()

        start_gather(0, 0, sem0)

        def pair(p, carry):
            g = 2 * p
            start_gather(g + 1, 1, sem1)
            drain_writeback(g, 0, sem0)

            @pl.when(g + 2 < ngroups)
            def _next():
                start_gather(g + 2, 0, sem0)

            drain_writeback(g + 1, 1, sem1)
            return carry

        lax.fori_loop(0, ngroups // 2, pair, 0)

    return body(idx_flat, pts_tab)


def kernel(s_xyz, xyz, s_points, nsample):
    del nsample
    B, N, _ = s_xyz.shape
    S = xyz.shape[1]
    D = s_points.shape[2]
    sT = jnp.transpose(s_xyz, (0, 2, 1))
    pts_tab = s_points.reshape(B * N, D)
    NSLAB = 4
    h = S // NSLAB
    gxns, outs = [], []
    for i in range(NSLAB):
        idx_i, gxn_i = _tc_topk(xyz[:, i * h:(i + 1) * h], sT, s_xyz)
        outs.append(_sc_gather(idx_i.reshape(-1), pts_tab))
        gxns.append(gxn_i.reshape(B, h, K, 3))
    gxyzn = jnp.concatenate(gxns, axis=1)
    out_p = jnp.concatenate(
        [o.reshape(B, h, K, D) for o in outs], axis=1)
    return jnp.concatenate([gxyzn, out_p], axis=-1)


# speedup vs baseline: 1.0407x; 1.0407x over previous
import functools

import jax
import jax.numpy as jnp
from jax import lax
from jax.experimental import pallas as pl
from jax.experimental.pallas import tpu as pltpu
from jax.experimental.pallas import tpu_sc as plsc

K = 16
TS = 256
BIG = 3.0e38


def _topk_body(xyz_ref, sxyzT_ref, sxyz_ref, idx_ref, gxn_ref):
    b = pl.program_id(0)
    x = xyz_ref[0]
    sT = sxyzT_ref[0]
    smat = sxyz_ref[0]
    n = sT.shape[1]
    s0, s1, s2 = sT[0:1, :], sT[1:2, :], sT[2:3, :]
    x0, x1, x2 = x[:, 0:1], x[:, 1:2], x[:, 2:3]
    ssq = (s0 * s0 + s1 * s1) + s2 * s2
    xsq = (x0 * x0 + x1 * x1) + x2 * x2
    mm = lax.dot_general(
        x.astype(jnp.bfloat16), sT.astype(jnp.bfloat16),
        (((1,), (0,)), ((), ())), preferred_element_type=jnp.float32)
    d = (-2.0 * mm + xsq) + ssq
    iota = lax.broadcasted_iota(jnp.int32, (TS, n), 1)
    smat_hi = smat.astype(jnp.bfloat16)
    smat_lo = (smat - smat_hi.astype(jnp.float32)).astype(jnp.bfloat16)
    iota_n = lax.broadcasted_iota(jnp.int32, (n, 1), 0)
    ext = jnp.concatenate([
        smat_hi, smat_lo,
        (iota_n // 64).astype(jnp.bfloat16),
        (iota_n % 64).astype(jnp.bfloat16),
        jnp.full((n, 1), 1.0, jnp.bfloat16),
    ], axis=1)
    d0 = d
    sels, gxs = [], []
    tiecnt = jnp.zeros((TS, 1), jnp.float32)
    for _ in range(K):
        m = jnp.min(d, axis=1, keepdims=True)
        eq = d == m
        onehot = jnp.where(eq, 1.0, 0.0).astype(jnp.bfloat16)
        ghl = lax.dot_general(onehot, ext, (((1,), (0,)), ((), ())),
                              preferred_element_type=jnp.float32)
        sel = (ghl[:, 6:7] * 64.0 + ghl[:, 7:8]).astype(jnp.int32)
        tiecnt = jnp.maximum(tiecnt, ghl[:, 8:9])
        sels.append(sel)
        gxs.append((ghl[:, 0:3] + ghl[:, 3:6]) - x)
        d = jnp.where(iota == sel, BIG, d)
    idx_fast = jnp.concatenate(sels, axis=1)
    gxn_fast = jnp.concatenate(gxs, axis=1)

    def _exact_topk():
        dd = d0
        sels_e, gxs_e = [], []
        for _ in range(K):
            me = jnp.min(dd, axis=1, keepdims=True)
            cand = jnp.where(dd == me, iota, n)
            sel_e = jnp.min(cand, axis=1, keepdims=True)
            hit = iota == sel_e
            gx_e = jnp.concatenate(
                [jnp.sum(jnp.where(hit, sc, 0.0), axis=1, keepdims=True)
                 for sc in (s0, s1, s2)], axis=1)
            sels_e.append(sel_e)
            gxs_e.append(gx_e - x)
            dd = jnp.where(hit, BIG, dd)
        return (jnp.concatenate(sels_e, axis=1),
                jnp.concatenate(gxs_e, axis=1))

    idx, gxn = lax.cond(jnp.max(tiecnt) >= 1.5, _exact_topk,
                        lambda: (idx_fast, gxn_fast))
    idx_ref[0] = idx + b * n
    gxn_ref[0] = gxn


def _tc_topk(xyz, s_xyzT, s_xyz):
    B, S, _ = xyz.shape
    N = s_xyzT.shape[2]
    return pl.pallas_call(
        _topk_body,
        grid=(B, S // TS),
        in_specs=[
            pl.BlockSpec((1, TS, 3), lambda b, t: (b, t, 0)),
            pl.BlockSpec((1, 3, N), lambda b, t: (b, 0, 0)),
            pl.BlockSpec((1, N, 3), lambda b, t: (b, 0, 0)),
        ],
        out_specs=[
            pl.BlockSpec((1, TS, K), lambda b, t: (b, t, 0)),
            pl.BlockSpec((1, TS, K * 3), lambda b, t: (b, t, 0)),
        ],
        out_shape=[
            jax.ShapeDtypeStruct((B, S, K), jnp.int32),
            jax.ShapeDtypeStruct((B, S, K * 3), jnp.float32),
        ],
    )(xyz, s_xyzT, s_xyz)


def _sc_gather(idx_flat, pts_tab):
    QK = idx_flat.shape[0]
    D = pts_tab.shape[1]
    info = plsc.get_sparse_core_info()
    NC, NS = info.num_cores, info.num_subcores
    NW = NC * NS
    per_w = QK // NW
    G = 128
    ngroups = per_w // G
    mesh = plsc.VectorSubcoreMesh(core_axis_name="c", subcore_axis_name="s")

    @functools.partial(
        pl.kernel,
        out_type=jax.ShapeDtypeStruct((QK, D), jnp.float32),
        mesh=mesh,
        scratch_types=[
            pltpu.VMEM((2, G), jnp.int32),
            pltpu.VMEM((2, G, D), jnp.float32),
            pltpu.SemaphoreType.DMA,
            pltpu.SemaphoreType.DMA,
            pltpu.SemaphoreType.DMA,
        ],
    )
    def body(idx_hbm, pts_hbm, outp_hbm, idx_v, pts_v, sem0, sem1, osem):
        wid = lax.axis_index("s") * NC + lax.axis_index("c")
        base = wid * per_w

        def start_gather(g, slot, sem):
            pltpu.sync_copy(idx_hbm.at[pl.ds(base + g * G, G)],
                            idx_v.at[slot])
            pltpu.async_copy(pts_hbm.at[idx_v.at[slot]], pts_v.at[slot], sem)

        def drain_writeback(g, slot, sem):
            pltpu.make_async_copy(pts_hbm.at[idx_v.at[slot]],
                                  pts_v.at[slot], sem).wait()
            pltpu.async_copy(pts_v.at[slot],
                             outp_hbm.at[pl.ds(base + g * G, G)], osem).wait()

        start_gather(0, 0, sem0)

        def pair(p, carry):
            g = 2 * p
            start_gather(g + 1, 1, sem1)
            drain_writeback(g, 0, sem0)

            @pl.when(g + 2 < ngroups)
            def _next():
                start_gather(g + 2, 0, sem0)

            drain_writeback(g + 1, 1, sem1)
            return carry

        lax.fori_loop(0, ngroups // 2, pair, 0)

    return body(idx_flat, pts_tab)


def kernel(s_xyz, xyz, s_points, nsample):
    del nsample
    B, N, _ = s_xyz.shape
    S = xyz.shape[1]
    D = s_points.shape[2]
    sT = jnp.transpose(s_xyz, (0, 2, 1))
    pts_tab = s_points.reshape(B * N, D)
    NSLAB = 2
    h = S // NSLAB
    gxns, outs = [], []
    for i in range(NSLAB):
        idx_i, gxn_i = _tc_topk(xyz[:, i * h:(i + 1) * h], sT, s_xyz)
        outs.append(_sc_gather(idx_i.reshape(-1), pts_tab))
        gxns.append(gxn_i.reshape(B, h, K, 3))
    gxyzn = jnp.concatenate(gxns, axis=1)
    out_p = jnp.concatenate(
        [o.reshape(B, h, K, D) for o in outs], axis=1)
    return jnp.concatenate([gxyzn, out_p], axis=-1)
